# baseline (speedup 1.0000x reference)
import functools

import jax
import jax.numpy as jnp
from jax import lax
from jax.experimental import pallas as pl
from jax.experimental.pallas import tpu as pltpu
from jax.experimental.pallas import tpu_sc as plsc

_D = 64
_B = 16384
_V = 100000
_LAMBDA = 0.001
_NC, _NS, _L = 2, 16, 16
_NW = _NC * _NS
_DPW = _D // _NW
_SCALE = _LAMBDA / _B

_T_OFF = (0, 33408, 66816)
_T_SZ = (33408, 33408, 33152)
_TAIL0 = 99968
_TAIL = 32
_TBUF = 33408
_NT = 3
_NU = _DPW * _NT

_mesh = plsc.VectorSubcoreMesh(core_axis_name="c", subcore_axis_name="s")


@functools.partial(
    pl.kernel,
    mesh=_mesh,
    out_type=jax.ShapeDtypeStruct((_NW * _L,), jnp.float32),
    compiler_params=pltpu.CompilerParams(
        use_tc_tiling_on_sc=True, needs_layout_passes=False),
    scratch_types=[
        pltpu.VMEM((_TBUF,), jnp.float32),
        pltpu.VMEM((_TBUF,), jnp.float32),
        pltpu.VMEM((_TAIL,), jnp.float32),
        pltpu.VMEM((_B,), jnp.float32),
        pltpu.VMEM((_B,), jnp.float32),
        pltpu.VMEM((_B,), jnp.int32),
        pltpu.VMEM((_L,), jnp.float32),
        pltpu.SemaphoreType.DMA,
        pltpu.SemaphoreType.DMA,
        pltpu.SemaphoreType.DMA,
    ],
)
def _center_loss_sc(ft_hbm, lab_hbm, ct_hbm, out_hbm,
                    crow_a, crow_b, tail_v, frow_0, frow_1, lab_v, out_v,
                    csem_a, csem_b, fsem):
    wid = lax.axis_index("s") * _NC + lax.axis_index("c")
    crows = (crow_a, crow_b)
    frows = (frow_0, frow_1)
    csems = (csem_a, csem_b)

    def crow_copy(u):
        d = wid * _DPW + u // _NT
        t = u % _NT
        return pltpu.async_copy(
            ct_hbm.at[d, pl.ds(_T_OFF[t], _T_SZ[t])],
            crows[u % 2].at[pl.ds(0, _T_SZ[t])], csems[u % 2])

    lab_cp = pltpu.async_copy(lab_hbm, lab_v, fsem)
    f0_cp = pltpu.async_copy(ft_hbm.at[wid * _DPW], frow_0, fsem)
    cps = {0: crow_copy(0)}
    lab_cp.wait()
    f0_cp.wait()

    accs = tuple(jnp.zeros((_L,), jnp.float32) for _ in range(4))
    f1_cp = None
    for u in range(_NU):
        cps[u].wait()
        if u + 1 < _NU:
            cps[u + 1] = crow_copy(u + 1)
        if u == _NT - 2:
            f1_cp = pltpu.async_copy(ft_hbm.at[wid * _DPW + 1], frow_1, fsem)
        if u == _NT:
            f1_cp.wait()
        if u % _NT == _NT - 1:
            d = wid * _DPW + u // _NT
            pltpu.sync_copy(ct_hbm.at[d, pl.ds(_TAIL0, _TAIL)], tail_v)

        t = u % _NT
        c0 = _T_OFF[t]
        sz = _T_SZ[t]
        crow = crows[u % 2]
        frow = frows[u // _NT]

        last = t == _NT - 1

        @plsc.parallel_loop(0, _B // _L, 4, unroll=4, carry=accs)
        def body(k, a, c0=c0, sz=sz, crow=crow, frow=frow, last=last):
            out = list(a)
            for j in range(4):
                base = (k + j) * _L
                idx = lab_v[pl.ds(base, _L)]
                f = frow[pl.ds(base, _L)]
                rel = idx - c0
                inb = plsc.bitcast(rel, jnp.uint32) < jnp.uint32(sz)
                c = plsc.load_gather(crow, [rel], mask=inb)
                df = jnp.where(inb, f - c, 0.0)
                acc_j = out[j] + df * df
                if last:
                    rel2 = idx - _TAIL0
                    inb2 = plsc.bitcast(rel2, jnp.uint32) < jnp.uint32(_TAIL)
                    c2 = plsc.load_gather(tail_v, [rel2], mask=inb2)
                    df2 = jnp.where(inb2, f - c2, 0.0)
                    acc_j = acc_j + df2 * df2
                out[j] = acc_j
            return tuple(out)

        accs = body

    acc = (accs[0] + accs[1]) + (accs[2] + accs[3])
    out_v[...] = acc * _SCALE
    pltpu.sync_copy(out_v, out_hbm.at[pl.ds(wid * _L, _L)])


def kernel(features, labels, centers):
    ft = features.T
    ct = centers.T
    lab = labels.astype(jnp.int32)
    partials = _center_loss_sc(ft, lab, ct)
    return jnp.sum(partials)


# speedup vs baseline: 1.0863x; 1.0863x over previous
import functools

import jax
import jax.numpy as jnp
from jax import lax
from jax.experimental import pallas as pl
from jax.experimental.pallas import tpu as pltpu
from jax.experimental.pallas import tpu_sc as plsc

_D = 64
_B = 16384
_V = 100000
_LAMBDA = 0.001
_NC, _NS, _L = 2, 16, 16
_NW = _NC * _NS
_DPW = _D // _NW
_SCALE = _LAMBDA / _B

_T_OFF = (0, 33408, 66816)
_T_SZ = (33408, 33408, 33152)
_TAIL0 = 99968
_TAIL = 32
_TBUF = 33408
_NT = 3
_NU = _DPW * _NT

_mesh = plsc.VectorSubcoreMesh(core_axis_name="c", subcore_axis_name="s")


@functools.partial(
    pl.kernel,
    mesh=_mesh,
    out_type=jax.ShapeDtypeStruct((_NW * _L,), jnp.float32),
    compiler_params=pltpu.CompilerParams(
        use_tc_tiling_on_sc=True, needs_layout_passes=False),
    scratch_types=[
        pltpu.VMEM((_TBUF,), jnp.float32),
        pltpu.VMEM((_TBUF,), jnp.float32),
        pltpu.VMEM((_TAIL,), jnp.float32),
        pltpu.VMEM((_B,), jnp.float32),
        pltpu.VMEM((_B,), jnp.float32),
        pltpu.VMEM((_B,), jnp.int32),
        pltpu.VMEM((_L,), jnp.float32),
        pltpu.SemaphoreType.DMA,
        pltpu.SemaphoreType.DMA,
        pltpu.SemaphoreType.DMA,
    ],
)
def _center_loss_sc(ft_hbm, lab_hbm, ct_hbm, out_hbm,
                    crow_a, crow_b, tail_v, frow_0, frow_1, lab_v, out_v,
                    csem_a, csem_b, fsem):
    wid = lax.axis_index("s") * _NC + lax.axis_index("c")
    crows = (crow_a, crow_b)
    frows = (frow_0, frow_1)
    csems = (csem_a, csem_b)

    def crow_copy(u):
        d = wid * _DPW + u // _NT
        t = u % _NT
        return pltpu.async_copy(
            ct_hbm.at[d, pl.ds(_T_OFF[t], _T_SZ[t])],
            crows[u % 2].at[pl.ds(0, _T_SZ[t])], csems[u % 2])

    lab_cp = pltpu.async_copy(lab_hbm, lab_v, fsem)
    f0_cp = pltpu.async_copy(ft_hbm.at[wid * _DPW], frow_0, fsem)
    cps = {0: crow_copy(0)}
    lab_cp.wait()
    f0_cp.wait()

    accs = tuple(jnp.zeros((_L,), jnp.float32) for _ in range(4))
    f1_cp = None
    for u in range(_NU):
        cps[u].wait()
        if u + 1 < _NU:
            cps[u + 1] = crow_copy(u + 1)
        if u == _NT - 2:
            f1_cp = pltpu.async_copy(ft_hbm.at[wid * _DPW + 1], frow_1, fsem)
        if u == _NT:
            f1_cp.wait()
        if u % _NT == _NT - 1:
            d = wid * _DPW + u // _NT
            pltpu.sync_copy(ct_hbm.at[d, pl.ds(_TAIL0, _TAIL)], tail_v)

        t = u % _NT
        c0 = _T_OFF[t]
        sz = _T_SZ[t]
        crow = crows[u % 2]
        frow = frows[u // _NT]

        last = t == _NT - 1

        @plsc.parallel_loop(0, _B // _L, 4, unroll=2, carry=accs)
        def body(k, a, c0=c0, sz=sz, crow=crow, frow=frow, last=last):
            out = list(a)
            for j in range(4):
                base = (k + j) * _L
                idx = lab_v[pl.ds(base, _L)]
                f = frow[pl.ds(base, _L)]
                rel = idx - c0
                inb = plsc.bitcast(rel, jnp.uint32) < jnp.uint32(sz)
                c = plsc.load_gather(crow, [rel], mask=inb)
                df = jnp.where(inb, f - c, 0.0)
                acc_j = out[j] + df * df
                if last:
                    rel2 = idx - _TAIL0
                    inb2 = plsc.bitcast(rel2, jnp.uint32) < jnp.uint32(_TAIL)
                    c2 = plsc.load_gather(tail_v, [rel2], mask=inb2)
                    df2 = jnp.where(inb2, f - c2, 0.0)
                    acc_j = acc_j + df2 * df2
                out[j] = acc_j
            return tuple(out)

        accs = body

    acc = (accs[0] + accs[1]) + (accs[2] + accs[3])
    out_v[...] = acc * _SCALE
    pltpu.sync_copy(out_v, out_hbm.at[pl.ds(wid * _L, _L)])


def kernel(features, labels, centers):
    ft = features.T
    ct = centers.T
    lab = labels.astype(jnp.int32)
    partials = _center_loss_sc(ft, lab, ct)
    return jnp.sum(partials)
